# baseline (speedup 1.0000x reference)
import functools

import jax
import jax.numpy as jnp
from jax.experimental import pallas as pl
from jax.experimental.pallas import tpu as pltpu
from jax.experimental.pallas import tpu_sc as plsc

_B = 2
_P = 12000
_N = 100
_CI = 10
_CO = 64
_ROW = 128
_G = 500
_CELLS = _G * _G

_TP = 96
_NT = _P // _TP

_CHUNK = 120
_NCHUNK = _P // _CHUNK

_MZ = 25000
_NMZ = _CELLS // _MZ


def _encode_body(u_ref, w_ref, pmax_ref, mom_ref):
    u = u_ref[0]
    w = w_ref[...]
    x = jax.lax.dot_general(w, u, (((1,), (0,)), ((), ())),
                            preferred_element_type=jnp.float32)
    sx = jnp.sum(x, axis=1)
    sx2 = jnp.sum(x * x, axis=1)
    mom_ref[0, 0] = jnp.stack([sx, sx2], axis=0)
    x3 = x.reshape(_CO, _TP, _N)
    pmax = jnp.max(x3, axis=-1)
    pmax_ref[0] = pmax.T


def _affine_body(pmax_ref, ab_ref, pidx_ref, ep_ref, ling_ref, linl_ref):
    b = pl.program_id(0)
    a = ab_ref[0, 0]
    bb = ab_ref[0, 1]
    ep = jnp.maximum(pmax_ref[0] * a[None, :] + bb[None, :], 0.0)
    ep_ref[0] = jnp.concatenate(
        [ep, jnp.zeros((_P, _ROW - _CO), jnp.float32)], axis=1)
    r = pidx_ref[0, 0, :]
    c = pidx_ref[0, 1, :]
    lin = r * _G + c
    linl_ref[0, 0] = lin
    ling_ref[0, 0] = lin + b * _CELLS


def _bev_body(NC, ep_hbm, ling_hbm, linl_hbm, ones_hbm, zeros_hbm,
              bev_hbm, mark_hbm, idxg_v, idxl_v, rows_v, ones_v, zeros_v,
              mark_sh):
    wid = jax.lax.axis_index("s") * NC + jax.lax.axis_index("c")
    for b in range(_B):
        @pl.when(wid == b)
        def _scatter():
            pltpu.sync_copy(zeros_hbm, zeros_v)

            def zchunk(k, carry):
                pltpu.sync_copy(zeros_v, mark_sh.at[pl.ds(k * _MZ, _MZ)])
                return carry

            jax.lax.fori_loop(0, _NMZ, zchunk, 0)

            pltpu.sync_copy(ones_hbm, ones_v)
            pltpu.sync_copy(ling_hbm.at[b], idxg_v)
            pltpu.sync_copy(linl_hbm.at[b], idxl_v)

            def chunk(j, carry):
                pltpu.sync_copy(
                    ep_hbm.at[pl.ds(b * _P + j * _CHUNK, _CHUNK)], rows_v)
                pltpu.sync_copy(rows_v, bev_hbm.at[idxg_v.at[j]])
                pltpu.sync_copy(ones_v, mark_sh.at[idxl_v.at[j]], add=True)
                return carry

            jax.lax.fori_loop(0, _NCHUNK, chunk, 0)

            def mdump(k, carry):
                pltpu.sync_copy(mark_sh.at[pl.ds(k * _MZ, _MZ)], zeros_v)
                pltpu.sync_copy(
                    zeros_v, mark_hbm.at[pl.ds(b * _CELLS + k * _MZ, _MZ)])
                return carry

            jax.lax.fori_loop(0, _NMZ, mdump, 0)


def _relayout_body(cm_ref, mark_ref, out_ref):
    m = mark_ref[0, 0]
    v = cm_ref[0, :, :_CO]
    x = jnp.where(m != 0, v.T, 0.0)
    out_ref[0, :, 0] = x.reshape(_CO, 4, _G)


def kernel(input_tensor, pillar_idxs, conv_w, bn_gamma, bn_beta):
    u_flat = input_tensor.reshape(_B, _CI, _P * _N)

    pmax_cm, mom = pl.pallas_call(
        _encode_body,
        grid=(_B, _NT),
        in_specs=[
            pl.BlockSpec((1, _CI, _TP * _N), lambda b, t: (b, 0, t)),
            pl.BlockSpec((_CO, _CI), lambda b, t: (0, 0)),
        ],
        out_specs=[
            pl.BlockSpec((1, _TP, _CO), lambda b, t: (b, t, 0)),
            pl.BlockSpec((1, 1, 2, _CO), lambda b, t: (b, t, 0, 0)),
        ],
        out_shape=[
            jax.ShapeDtypeStruct((_B, _P, _CO), jnp.float32),
            jax.ShapeDtypeStruct((_B, _NT, 2, _CO), jnp.float32),
        ],
    )(u_flat, conv_w)

    sums = jnp.sum(mom, axis=(0, 1))
    n_tot = float(_B * _P * _N)
    mean = sums[0] / n_tot
    var = sums[1] / n_tot - mean * mean
    a = bn_gamma * jax.lax.rsqrt(var + 1e-5)
    b = bn_beta - mean * a
    ab = jnp.stack([a, b], axis=0)[None]

    pidx_t = jnp.swapaxes(pillar_idxs, 1, 2)

    ep_cm, ling, linl = pl.pallas_call(
        _affine_body,
        grid=(_B,),
        in_specs=[
            pl.BlockSpec((1, _P, _CO), lambda b: (b, 0, 0)),
            pl.BlockSpec((1, 2, _CO), lambda b: (0, 0, 0)),
            pl.BlockSpec((1, 2, _P), lambda b: (b, 0, 0)),
        ],
        out_specs=[
            pl.BlockSpec((1, _P, _ROW), lambda b: (b, 0, 0)),
            pl.BlockSpec((1, 1, _P), lambda b: (b, 0, 0)),
            pl.BlockSpec((1, 1, _P), lambda b: (b, 0, 0)),
        ],
        out_shape=[
            jax.ShapeDtypeStruct((_B, _P, _ROW), jnp.float32),
            jax.ShapeDtypeStruct((_B, 1, _P), jnp.int32),
            jax.ShapeDtypeStruct((_B, 1, _P), jnp.int32),
        ],
    )(pmax_cm, ab, pidx_t)

    ep_flat = ep_cm.reshape(_B * _P, _ROW)
    ling3 = ling.reshape(_B, _NCHUNK, _CHUNK)
    linl3 = linl.reshape(_B, _NCHUNK, _CHUNK)
    ones_src = jnp.ones((_CHUNK,), dtype=jnp.int32)
    zeros_src = jnp.zeros((_MZ,), dtype=jnp.int32)

    info = plsc.get_sparse_core_info()
    NC = info.num_cores
    mesh = plsc.VectorSubcoreMesh(core_axis_name="c", subcore_axis_name="s")

    bev_cm, mark = pl.kernel(
        functools.partial(_bev_body, NC),
        mesh=mesh,
        out_type=[
            jax.ShapeDtypeStruct((_B * _CELLS, _ROW), jnp.float32),
            jax.ShapeDtypeStruct((_B * _CELLS,), jnp.int32),
        ],
        scratch_types=[
            pltpu.VMEM((_NCHUNK, _CHUNK), jnp.int32),
            pltpu.VMEM((_NCHUNK, _CHUNK), jnp.int32),
            pltpu.VMEM((_CHUNK, _ROW), jnp.float32),
            pltpu.VMEM((_CHUNK,), jnp.int32),
            pltpu.VMEM((_MZ,), jnp.int32),
            pltpu.VMEM_SHARED((_CELLS,), jnp.int32),
        ],
    )(ep_flat, ling3, linl3, ones_src, zeros_src)

    bev_cm3 = bev_cm.reshape(_B, _CELLS, _ROW)
    mark4 = mark.reshape(_B, 125, 1, 2000)

    out5 = pl.pallas_call(
        _relayout_body,
        grid=(_B, 125),
        in_specs=[
            pl.BlockSpec((1, 2000, _ROW), lambda b, r: (b, r, 0)),
            pl.BlockSpec((1, 1, 1, 2000), lambda b, r: (b, r, 0, 0)),
        ],
        out_specs=pl.BlockSpec((1, _CO, 1, 4, _G), lambda b, r: (b, 0, r, 0, 0)),
        out_shape=jax.ShapeDtypeStruct((_B, _CO, 125, 4, _G), jnp.float32),
    )(bev_cm3, mark4)

    return out5.reshape(_B, _CO, _G, _G)


# speedup vs baseline: 1.0154x; 1.0154x over previous
import functools

import jax
import jax.numpy as jnp
from jax.experimental import pallas as pl
from jax.experimental.pallas import tpu as pltpu
from jax.experimental.pallas import tpu_sc as plsc

_B = 2
_P = 12000
_N = 100
_CI = 10
_CO = 64
_ROW = 128
_G = 500
_CELLS = _G * _G

_TP = 96
_NT = _P // _TP

_CHUNK = 120
_NCHUNK = _P // _CHUNK

_MZ = 25000
_NMZ = _CELLS // _MZ


def _encode_body(u_ref, w_ref, pmax_ref, mom_ref):
    u = u_ref[0]
    w = w_ref[...]
    x = jax.lax.dot_general(w, u, (((1,), (0,)), ((), ())),
                            preferred_element_type=jnp.float32)
    sx = jnp.sum(x, axis=1)
    sx2 = jnp.sum(x * x, axis=1)
    mom_ref[0, 0] = jnp.stack([sx, sx2], axis=0)
    x3 = x.reshape(_CO, _TP, _N)
    pmax = jnp.max(x3, axis=-1)
    pmax_ref[0] = pmax.T


def _affine_body(pmax_ref, ab_ref, pidx_ref, ep_ref, ling_ref, linl_ref):
    b = pl.program_id(0)
    a = ab_ref[0, 0]
    bb = ab_ref[0, 1]
    ep = jnp.maximum(pmax_ref[0] * a[None, :] + bb[None, :], 0.0)
    ep_ref[0] = jnp.concatenate(
        [ep, jnp.zeros((_P, _ROW - _CO), jnp.float32)], axis=1)
    r = pidx_ref[0, 0, :]
    c = pidx_ref[0, 1, :]
    lin = r * _G + c
    linl_ref[0, 0] = lin
    ling_ref[0, 0] = lin + b * _CELLS


def _bev_body(NC, ep_hbm, ling_hbm, linl_hbm, ones_hbm, zeros_hbm,
              bev_hbm, mark_hbm, idxg_v, idxl_v, rows_v, rows_w, ones_v,
              zeros_v, sem_a, sem_b, mark_sh):
    wid = jax.lax.axis_index("s") * NC + jax.lax.axis_index("c")
    for b in range(_B):
        @pl.when(wid == b)
        def _scatter():
            pltpu.sync_copy(zeros_hbm, zeros_v)

            def zchunk(k, carry):
                pltpu.sync_copy(zeros_v, mark_sh.at[pl.ds(k * _MZ, _MZ)])
                return carry

            jax.lax.fori_loop(0, _NMZ, zchunk, 0)

            pltpu.sync_copy(ones_hbm, ones_v)
            pltpu.sync_copy(ling_hbm.at[b], idxg_v)
            pltpu.sync_copy(linl_hbm.at[b], idxl_v)

            def ep_src(j):
                return ep_hbm.at[pl.ds(b * _P + j * _CHUNK, _CHUNK)]

            pltpu.async_copy(ep_src(0), rows_v, sem_a)

            def chunk2(t, carry):
                j0 = 2 * t
                j1 = 2 * t + 1
                pltpu.make_async_copy(ep_src(j0), rows_v, sem_a).wait()
                pltpu.async_copy(ep_src(j1), rows_w, sem_b)
                pltpu.sync_copy(rows_v, bev_hbm.at[idxg_v.at[j0]])
                pltpu.sync_copy(ones_v, mark_sh.at[idxl_v.at[j0]], add=True)
                pltpu.make_async_copy(ep_src(j1), rows_w, sem_b).wait()

                @pl.when(t < _NCHUNK // 2 - 1)
                def _pref():
                    pltpu.async_copy(ep_src(j1 + 1), rows_v, sem_a)

                pltpu.sync_copy(rows_w, bev_hbm.at[idxg_v.at[j1]])
                pltpu.sync_copy(ones_v, mark_sh.at[idxl_v.at[j1]], add=True)
                return carry

            jax.lax.fori_loop(0, _NCHUNK // 2, chunk2, 0)

            def mdump(k, carry):
                pltpu.sync_copy(mark_sh.at[pl.ds(k * _MZ, _MZ)], zeros_v)
                pltpu.sync_copy(
                    zeros_v, mark_hbm.at[pl.ds(b * _CELLS + k * _MZ, _MZ)])
                return carry

            jax.lax.fori_loop(0, _NMZ, mdump, 0)


def _relayout_body(cm_ref, mark_ref, out_ref):
    m = mark_ref[0, 0]
    v = cm_ref[0, :, :_CO]
    x = jnp.where(m != 0, v.T, 0.0)
    out_ref[0, :, 0] = x.reshape(_CO, 4, _G)


def kernel(input_tensor, pillar_idxs, conv_w, bn_gamma, bn_beta):
    u_flat = input_tensor.reshape(_B, _CI, _P * _N)

    pmax_cm, mom = pl.pallas_call(
        _encode_body,
        grid=(_B, _NT),
        in_specs=[
            pl.BlockSpec((1, _CI, _TP * _N), lambda b, t: (b, 0, t)),
            pl.BlockSpec((_CO, _CI), lambda b, t: (0, 0)),
        ],
        out_specs=[
            pl.BlockSpec((1, _TP, _CO), lambda b, t: (b, t, 0)),
            pl.BlockSpec((1, 1, 2, _CO), lambda b, t: (b, t, 0, 0)),
        ],
        out_shape=[
            jax.ShapeDtypeStruct((_B, _P, _CO), jnp.float32),
            jax.ShapeDtypeStruct((_B, _NT, 2, _CO), jnp.float32),
        ],
    )(u_flat, conv_w)

    sums = jnp.sum(mom, axis=(0, 1))
    n_tot = float(_B * _P * _N)
    mean = sums[0] / n_tot
    var = sums[1] / n_tot - mean * mean
    a = bn_gamma * jax.lax.rsqrt(var + 1e-5)
    b = bn_beta - mean * a
    ab = jnp.stack([a, b], axis=0)[None]

    pidx_t = jnp.swapaxes(pillar_idxs, 1, 2)

    ep_cm, ling, linl = pl.pallas_call(
        _affine_body,
        grid=(_B,),
        in_specs=[
            pl.BlockSpec((1, _P, _CO), lambda b: (b, 0, 0)),
            pl.BlockSpec((1, 2, _CO), lambda b: (0, 0, 0)),
            pl.BlockSpec((1, 2, _P), lambda b: (b, 0, 0)),
        ],
        out_specs=[
            pl.BlockSpec((1, _P, _ROW), lambda b: (b, 0, 0)),
            pl.BlockSpec((1, 1, _P), lambda b: (b, 0, 0)),
            pl.BlockSpec((1, 1, _P), lambda b: (b, 0, 0)),
        ],
        out_shape=[
            jax.ShapeDtypeStruct((_B, _P, _ROW), jnp.float32),
            jax.ShapeDtypeStruct((_B, 1, _P), jnp.int32),
            jax.ShapeDtypeStruct((_B, 1, _P), jnp.int32),
        ],
    )(pmax_cm, ab, pidx_t)

    ep_flat = ep_cm.reshape(_B * _P, _ROW)
    ling3 = ling.reshape(_B, _NCHUNK, _CHUNK)
    linl3 = linl.reshape(_B, _NCHUNK, _CHUNK)
    ones_src = jnp.ones((_CHUNK,), dtype=jnp.int32)
    zeros_src = jnp.zeros((_MZ,), dtype=jnp.int32)

    info = plsc.get_sparse_core_info()
    NC = info.num_cores
    mesh = plsc.VectorSubcoreMesh(core_axis_name="c", subcore_axis_name="s")

    bev_cm, mark = pl.kernel(
        functools.partial(_bev_body, NC),
        mesh=mesh,
        out_type=[
            jax.ShapeDtypeStruct((_B * _CELLS, _ROW), jnp.float32),
            jax.ShapeDtypeStruct((_B * _CELLS,), jnp.int32),
        ],
        scratch_types=[
            pltpu.VMEM((_NCHUNK, _CHUNK), jnp.int32),
            pltpu.VMEM((_NCHUNK, _CHUNK), jnp.int32),
            pltpu.VMEM((_CHUNK, _ROW), jnp.float32),
            pltpu.VMEM((_CHUNK, _ROW), jnp.float32),
            pltpu.VMEM((_CHUNK,), jnp.int32),
            pltpu.VMEM((_MZ,), jnp.int32),
            pltpu.SemaphoreType.DMA,
            pltpu.SemaphoreType.DMA,
            pltpu.VMEM_SHARED((_CELLS,), jnp.int32),
        ],
    )(ep_flat, ling3, linl3, ones_src, zeros_src)

    bev_cm3 = bev_cm.reshape(_B, _CELLS, _ROW)
    mark4 = mark.reshape(_B, 125, 1, 2000)

    out5 = pl.pallas_call(
        _relayout_body,
        grid=(_B, 125),
        in_specs=[
            pl.BlockSpec((1, 2000, _ROW), lambda b, r: (b, r, 0)),
            pl.BlockSpec((1, 1, 1, 2000), lambda b, r: (b, r, 0, 0)),
        ],
        out_specs=pl.BlockSpec((1, _CO, 1, 4, _G), lambda b, r: (b, 0, r, 0, 0)),
        out_shape=jax.ShapeDtypeStruct((_B, _CO, 125, 4, _G), jnp.float32),
    )(bev_cm3, mark4)

    return out5.reshape(_B, _CO, _G, _G)


# speedup vs baseline: 1.0160x; 1.0006x over previous
import functools

import jax
import jax.numpy as jnp
from jax.experimental import pallas as pl
from jax.experimental.pallas import tpu as pltpu
from jax.experimental.pallas import tpu_sc as plsc

_B = 2
_P = 12000
_N = 100
_CI = 10
_CO = 64
_ROW = 128
_G = 500
_CELLS = _G * _G

_TP = 96
_NT = _P // _TP

_CHUNK = 120
_NCHUNK = _P // _CHUNK

_MZ = 25000
_NMZ = _CELLS // _MZ


def _encode_body(u_ref, w_ref, pmax_ref, mom_ref):
    u = u_ref[0]
    w = w_ref[...]
    x = jax.lax.dot_general(w, u, (((1,), (0,)), ((), ())),
                            preferred_element_type=jnp.float32)
    sx = jnp.sum(x, axis=1)
    sx2 = jnp.sum(x * x, axis=1)
    mom_ref[0, 0] = jnp.stack([sx, sx2], axis=0)
    x3 = x.reshape(_CO, _TP, _N)
    pmax = jnp.max(x3, axis=-1)
    pmax_ref[0] = pmax.T


def _affine_body(pmax_ref, ab_ref, pidx_ref, ep_ref, ling_ref, linl_ref):
    b = pl.program_id(0)
    a = ab_ref[0, 0]
    bb = ab_ref[0, 1]
    ep = jnp.maximum(pmax_ref[0] * a[None, :] + bb[None, :], 0.0)
    ep_ref[0] = jnp.concatenate(
        [ep, jnp.zeros((_P, _ROW - _CO), jnp.float32)], axis=1)
    r = pidx_ref[0, 0, :]
    c = pidx_ref[0, 1, :]
    lin = r * _G + c
    linl_ref[0, 0] = lin
    ling_ref[0, 0] = lin + b * _CELLS


def _bev_body(NC, ep_hbm, ling_hbm, linl_hbm, ones_hbm, zeros_hbm,
              bev_hbm, mark_hbm, idxg_v, idxl_v, rows_v, rows_w, ones_v,
              zeros_v, sem_a, sem_b, sem_m, mark_sh):
    wid = jax.lax.axis_index("s") * NC + jax.lax.axis_index("c")
    for b in range(_B):
        @pl.when(wid == b)
        def _scatter():
            pltpu.sync_copy(zeros_hbm, zeros_v)

            def zchunk(k, carry):
                pltpu.sync_copy(zeros_v, mark_sh.at[pl.ds(k * _MZ, _MZ)])
                return carry

            jax.lax.fori_loop(0, _NMZ, zchunk, 0)

            pltpu.sync_copy(ones_hbm, ones_v)
            pltpu.sync_copy(ling_hbm.at[b], idxg_v)
            pltpu.sync_copy(linl_hbm.at[b], idxl_v)

            def ep_src(j):
                return ep_hbm.at[pl.ds(b * _P + j * _CHUNK, _CHUNK)]

            pltpu.async_copy(ep_src(0), rows_v, sem_a)

            def chunk2(t, carry):
                j0 = 2 * t
                j1 = 2 * t + 1
                pltpu.make_async_copy(ep_src(j0), rows_v, sem_a).wait()
                pltpu.async_copy(ep_src(j1), rows_w, sem_b)
                pltpu.sync_copy(rows_v, bev_hbm.at[idxg_v.at[j0]])
                pltpu.async_copy(ones_v, mark_sh.at[idxl_v.at[j0]], sem_m,
                                 add=True)
                pltpu.make_async_copy(ep_src(j1), rows_w, sem_b).wait()

                @pl.when(t < _NCHUNK // 2 - 1)
                def _pref():
                    pltpu.async_copy(ep_src(j1 + 1), rows_v, sem_a)

                pltpu.sync_copy(rows_w, bev_hbm.at[idxg_v.at[j1]])
                pltpu.async_copy(ones_v, mark_sh.at[idxl_v.at[j1]], sem_m,
                                 add=True)
                return carry

            jax.lax.fori_loop(0, _NCHUNK // 2, chunk2, 0)

            def mdrain(j, carry):
                pltpu.make_async_copy(
                    ones_v, mark_sh.at[idxl_v.at[j]], sem_m).wait()
                return carry

            jax.lax.fori_loop(0, _NCHUNK, mdrain, 0)

            def mdump(k, carry):
                pltpu.sync_copy(mark_sh.at[pl.ds(k * _MZ, _MZ)], zeros_v)
                pltpu.sync_copy(
                    zeros_v, mark_hbm.at[pl.ds(b * _CELLS + k * _MZ, _MZ)])
                return carry

            jax.lax.fori_loop(0, _NMZ, mdump, 0)


def _relayout_body(cm_ref, mark_ref, out_ref):
    m = mark_ref[0, 0]
    v = cm_ref[0, :, :_CO]
    x = jnp.where(m != 0, v.T, 0.0)
    out_ref[0, :, 0] = x.reshape(_CO, 4, _G)


def kernel(input_tensor, pillar_idxs, conv_w, bn_gamma, bn_beta):
    u_flat = input_tensor.reshape(_B, _CI, _P * _N)

    pmax_cm, mom = pl.pallas_call(
        _encode_body,
        grid=(_B, _NT),
        in_specs=[
            pl.BlockSpec((1, _CI, _TP * _N), lambda b, t: (b, 0, t)),
            pl.BlockSpec((_CO, _CI), lambda b, t: (0, 0)),
        ],
        out_specs=[
            pl.BlockSpec((1, _TP, _CO), lambda b, t: (b, t, 0)),
            pl.BlockSpec((1, 1, 2, _CO), lambda b, t: (b, t, 0, 0)),
        ],
        out_shape=[
            jax.ShapeDtypeStruct((_B, _P, _CO), jnp.float32),
            jax.ShapeDtypeStruct((_B, _NT, 2, _CO), jnp.float32),
        ],
    )(u_flat, conv_w)

    sums = jnp.sum(mom, axis=(0, 1))
    n_tot = float(_B * _P * _N)
    mean = sums[0] / n_tot
    var = sums[1] / n_tot - mean * mean
    a = bn_gamma * jax.lax.rsqrt(var + 1e-5)
    b = bn_beta - mean * a
    ab = jnp.stack([a, b], axis=0)[None]

    pidx_t = jnp.swapaxes(pillar_idxs, 1, 2)

    ep_cm, ling, linl = pl.pallas_call(
        _affine_body,
        grid=(_B,),
        in_specs=[
            pl.BlockSpec((1, _P, _CO), lambda b: (b, 0, 0)),
            pl.BlockSpec((1, 2, _CO), lambda b: (0, 0, 0)),
            pl.BlockSpec((1, 2, _P), lambda b: (b, 0, 0)),
        ],
        out_specs=[
            pl.BlockSpec((1, _P, _ROW), lambda b: (b, 0, 0)),
            pl.BlockSpec((1, 1, _P), lambda b: (b, 0, 0)),
            pl.BlockSpec((1, 1, _P), lambda b: (b, 0, 0)),
        ],
        out_shape=[
            jax.ShapeDtypeStruct((_B, _P, _ROW), jnp.float32),
            jax.ShapeDtypeStruct((_B, 1, _P), jnp.int32),
            jax.ShapeDtypeStruct((_B, 1, _P), jnp.int32),
        ],
    )(pmax_cm, ab, pidx_t)

    ep_flat = ep_cm.reshape(_B * _P, _ROW)
    ling3 = ling.reshape(_B, _NCHUNK, _CHUNK)
    linl3 = linl.reshape(_B, _NCHUNK, _CHUNK)
    ones_src = jnp.ones((_CHUNK,), dtype=jnp.int32)
    zeros_src = jnp.zeros((_MZ,), dtype=jnp.int32)

    info = plsc.get_sparse_core_info()
    NC = info.num_cores
    mesh = plsc.VectorSubcoreMesh(core_axis_name="c", subcore_axis_name="s")

    bev_cm, mark = pl.kernel(
        functools.partial(_bev_body, NC),
        mesh=mesh,
        out_type=[
            jax.ShapeDtypeStruct((_B * _CELLS, _ROW), jnp.float32),
            jax.ShapeDtypeStruct((_B * _CELLS,), jnp.int32),
        ],
        scratch_types=[
            pltpu.VMEM((_NCHUNK, _CHUNK), jnp.int32),
            pltpu.VMEM((_NCHUNK, _CHUNK), jnp.int32),
            pltpu.VMEM((_CHUNK, _ROW), jnp.float32),
            pltpu.VMEM((_CHUNK, _ROW), jnp.float32),
            pltpu.VMEM((_CHUNK,), jnp.int32),
            pltpu.VMEM((_MZ,), jnp.int32),
            pltpu.SemaphoreType.DMA,
            pltpu.SemaphoreType.DMA,
            pltpu.SemaphoreType.DMA,
            pltpu.VMEM_SHARED((_CELLS,), jnp.int32),
        ],
    )(ep_flat, ling3, linl3, ones_src, zeros_src)

    bev_cm3 = bev_cm.reshape(_B, _CELLS, _ROW)
    mark4 = mark.reshape(_B, 125, 1, 2000)

    out5 = pl.pallas_call(
        _relayout_body,
        grid=(_B, 125),
        in_specs=[
            pl.BlockSpec((1, 2000, _ROW), lambda b, r: (b, r, 0)),
            pl.BlockSpec((1, 1, 1, 2000), lambda b, r: (b, r, 0, 0)),
        ],
        out_specs=pl.BlockSpec((1, _CO, 1, 4, _G), lambda b, r: (b, 0, r, 0, 0)),
        out_shape=jax.ShapeDtypeStruct((_B, _CO, 125, 4, _G), jnp.float32),
    )(bev_cm3, mark4)

    return out5.reshape(_B, _CO, _G, _G)
